# baseline (speedup 1.0000x reference)
import functools

import jax
import jax.numpy as jnp
from jax import lax
from jax.experimental import pallas as pl
from jax.experimental.pallas import tpu as pltpu
from jax.experimental.pallas import tpu_sc as plsc

B = 8
N = 2048
K = 20
CIN = 3
CP = 16
KTOP = 2 * K
ROWT = 256
EPS = 1e-5
NEG = float("-inf")

SC_NC = 2
SC_NS = 16
SC_NW = SC_NC * SC_NS
GCHUNK = 128
GB = 16
TOTAL_IDX = 2 * B * N * K
IDX_ROWS = TOTAL_IDX // GCHUNK
ROWS_PER_W = IDX_ROWS // SC_NW


def _topk_body(x8_ref, xt_ref, out_ref):
    xt = xt_ref[0]
    x0 = x8_ref[0, 0:1, :]
    x1 = x8_ref[0, 1:2, :]
    x2 = x8_ref[0, 2:3, :]

    xx_col = (x0 * x0 + x1 * x1) + x2 * x2
    t0 = xt[:, 0:1]
    t1 = xt[:, 1:2]
    t2 = xt[:, 2:3]
    xx_row = (t0 * t0 + t1 * t1) + t2 * t2
    def b16(v):
        return v.astype(jnp.bfloat16).astype(jnp.float32)
    g = (b16(t0) * b16(x0) + b16(t1) * b16(x1)) + b16(t2) * b16(x2)
    d = (-xx_col + 2.0 * g) - xx_row

    H = ROWT // 2
    lanes = lax.broadcasted_iota(jnp.int32, (H, N), 1)
    lanes40 = lax.broadcasted_iota(jnp.int32, (H, KTOP), 1)

    def half_init(dh):
        return (dh, jnp.full((H, 1), N, jnp.int32),
                jnp.zeros((H, KTOP), jnp.int32))

    def half_step(t, dh, j, acc):
        dh = jnp.where(lanes == j, NEG, dh)
        m = jnp.max(dh, axis=1, keepdims=True)
        j = jnp.min(jnp.where(dh == m, lanes, N), axis=1, keepdims=True)
        acc = jnp.where(lanes40 == t, j, acc)
        return dh, j, acc

    def step(t, carry):
        (da, ja, acca), (db, jb, accb) = carry
        return half_step(t, da, ja, acca), half_step(t, db, jb, accb)

    (_, _, acca), (_, _, accb) = lax.fori_loop(
        0, KTOP, step, (half_init(d[:H]), half_init(d[H:])))
    out_ref[0] = jnp.concatenate([acca, accb], axis=0)


def _topk40(x8, xt16):
    return pl.pallas_call(
        _topk_body,
        grid=(B, N // ROWT),
        in_specs=[
            pl.BlockSpec((1, 8, N), lambda b, i: (b, 0, 0)),
            pl.BlockSpec((1, ROWT, CP), lambda b, i: (b, i, 0)),
        ],
        out_specs=pl.BlockSpec((1, ROWT, KTOP), lambda b, i: (b, i, 0)),
        out_shape=jax.ShapeDtypeStruct((B, N, KTOP), jnp.int32),
    )(x8, xt16)


def _sc_gather(table, idx2d):
    mesh = plsc.VectorSubcoreMesh(core_axis_name="c", subcore_axis_name="s")

    @functools.partial(
        pl.kernel,
        mesh=mesh,
        compiler_params=pltpu.CompilerParams(use_tc_tiling_on_sc=False),
        out_type=jax.ShapeDtypeStruct((TOTAL_IDX, CP), jnp.float32),
        scratch_types=[
            pltpu.VMEM((ROWS_PER_W, GCHUNK), jnp.int32),
            pltpu.VMEM((GB * GCHUNK, CP), jnp.float32),
            pltpu.SemaphoreType.DMA,
        ],
    )
    def gk(table_hbm, idx_hbm, out_hbm, idx_v, rows_v, sem):
        wid = lax.axis_index("s") * SC_NC + lax.axis_index("c")
        base = wid * ROWS_PER_W
        pltpu.sync_copy(idx_hbm.at[pl.ds(base, ROWS_PER_W)], idx_v)

        def body(o, _):
            for b in range(GB):
                pltpu.async_copy(table_hbm.at[idx_v.at[o * GB + b]],
                                 rows_v.at[pl.ds(b * GCHUNK, GCHUNK)], sem)
            for b in range(GB):
                pltpu.make_async_copy(
                    table_hbm.at[idx_v.at[o * GB + b]],
                    rows_v.at[pl.ds(b * GCHUNK, GCHUNK)], sem).wait()
            pltpu.sync_copy(
                rows_v,
                out_hbm.at[pl.ds((base + o * GB) * GCHUNK, GB * GCHUNK)])
            return 0

        lax.fori_loop(0, ROWS_PER_W // GB, body, 0)

    return gk(table, idx2d)


def _pair_sum(v, axis):
    if axis == 1:
        left = jnp.concatenate([v[:, 1:], v[:, :1]], axis=1)
        right = jnp.concatenate([v[:, -1:], v[:, :-1]], axis=1)
    else:
        left = jnp.concatenate([v[1:, :], v[:1, :]], axis=0)
        right = jnp.concatenate([v[-1:, :], v[:-1, :]], axis=0)
    even = lax.broadcasted_iota(jnp.int32, v.shape, axis) % 2 == 0
    return v + jnp.where(even, left, right)


def _gn_lrelu(h, gamma, beta):
    n = h.shape[0] * 2
    s = jnp.sum(h, axis=0, keepdims=True)
    ss = jnp.sum(h * h, axis=0, keepdims=True)
    sg = _pair_sum(s, 1)
    ssg = _pair_sum(ss, 1)
    mean = sg / n
    var = ssg / n - mean * mean
    inv = lax.rsqrt(var + EPS)
    scale = gamma * inv
    shift = beta - mean * scale
    z = h * scale + shift
    return jnp.where(z >= 0, z, 0.2 * z)


def _fold512(v):
    v = v[:, :256] + v[:, 256:]
    v = v[:, :128] + v[:, 128:]
    return v[:, :64] + v[:, 64:]


def _tile8(v):
    return jnp.concatenate([v] * 8, axis=1)


def _gn_lrelu_packed(h, gamma, beta, nrows):
    n = nrows * 2
    s = _fold512(jnp.sum(h, axis=0, keepdims=True))
    ss = _fold512(jnp.sum(h * h, axis=0, keepdims=True))
    sg = _pair_sum(s, 1)
    ssg = _pair_sum(ss, 1)
    mean = sg / n
    var = ssg / n - mean * mean
    inv = lax.rsqrt(var + EPS)
    scale = gamma * inv
    shift = beta - mean * scale
    z = h * _tile8(scale) + _tile8(shift)
    return jnp.where(z >= 0, z, 0.2 * z)


NP = N // 8
RP = K * NP


def _branch_body(gath_ref, xta_ref, xtb_ref, w1_ref, w2_ref, wcat_ref,
                 gnp_ref, se1_ref, se2_ref, out_ref):
    g = gath_ref[0, 0]
    xa = xta_ref[0]
    xb = xtb_ref[0]

    e = (g.reshape(K, NP, 128) - xa[None] + xb[None]).reshape(RP, 128)
    h = jnp.dot(e, w1_ref[0], preferred_element_type=jnp.float32)
    h = _gn_lrelu_packed(h, gnp_ref[0, 0:1, :], gnp_ref[0, 1:2, :], K * N)
    h = jnp.dot(h, w2_ref[0], preferred_element_type=jnp.float32)
    h = _gn_lrelu_packed(h, gnp_ref[0, 2:3, :], gnp_ref[0, 3:4, :], K * N)

    h3 = h.reshape(K, NP, 512)
    mx = jnp.max(h3, axis=0)
    mn = jnp.sum(h3, axis=0) / K
    cat = jnp.concatenate([mx, mn], axis=1)
    c = jnp.dot(cat, wcat_ref[0], preferred_element_type=jnp.float32)
    c = _gn_lrelu_packed(c, gnp_ref[0, 4:5, :], gnp_ref[0, 5:6, :], N)

    y = _fold512(jnp.sum(c, axis=0, keepdims=True)) / N
    y1 = jax.nn.relu(lax.dot_general(y, se1_ref[0], (((1,), (1,)), ((), ())),
                                     preferred_element_type=jnp.float32))
    y2 = jax.nn.sigmoid(lax.dot_general(y1, se2_ref[0],
                                        (((1,), (1,)), ((), ())),
                                        preferred_element_type=jnp.float32))
    out_ref[0, 0] = c * _tile8(y2)


def _branches(gath, xta, xtb, w1s, w2s, wcats, gnps, se1s, se2s):
    return pl.pallas_call(
        _branch_body,
        grid=(2, B),
        in_specs=[
            pl.BlockSpec((1, 1, RP, 128), lambda r, b: (r, b, 0, 0)),
            pl.BlockSpec((1, NP, 128), lambda r, b: (b, 0, 0)),
            pl.BlockSpec((1, NP, 128), lambda r, b: (b, 0, 0)),
            pl.BlockSpec((1, 128, 512), lambda r, b: (r, 0, 0)),
            pl.BlockSpec((1, 512, 512), lambda r, b: (r, 0, 0)),
            pl.BlockSpec((1, 1024, 512), lambda r, b: (r, 0, 0)),
            pl.BlockSpec((1, 6, 64), lambda r, b: (r, 0, 0)),
            pl.BlockSpec((1, 4, 64), lambda r, b: (r, 0, 0)),
            pl.BlockSpec((1, 64, 4), lambda r, b: (r, 0, 0)),
        ],
        out_specs=pl.BlockSpec((1, 1, NP, 512), lambda r, b: (r, b, 0, 0)),
        out_shape=jax.ShapeDtypeStruct((2, B, NP, 512), jnp.float32),
    )(gath, xta, xtb, w1s, w2s, wcats, gnps, se1s, se2s)


def _final_body(b1_ref, b2_ref, wadd_ref, g_ref, bb_ref, out_ref):
    cat = jnp.concatenate([b1_ref[0, 0], b2_ref[0, 0]], axis=1)
    z = jnp.dot(cat, wadd_ref[...],
                preferred_element_type=jnp.float32)
    out_ref[0] = _gn_lrelu_packed(z, g_ref[...], bb_ref[...], N)


def _final(bout, waddbig, gg, gb):
    return pl.pallas_call(
        _final_body,
        grid=(B,),
        in_specs=[
            pl.BlockSpec((1, 1, NP, 512), lambda b: (0, b, 0, 0)),
            pl.BlockSpec((1, 1, NP, 512), lambda b: (1, b, 0, 0)),
            pl.BlockSpec((1024, 512), lambda b: (0, 0)),
            pl.BlockSpec((1, 64), lambda b: (0, 0)),
            pl.BlockSpec((1, 64), lambda b: (0, 0)),
        ],
        out_specs=pl.BlockSpec((1, NP, 512), lambda b: (b, 0, 0)),
        out_shape=jax.ShapeDtypeStruct((B, NP, 512), jnp.float32),
    )(bout, bout, waddbig, gg, gb)


def kernel(x, params):
    p = params
    xt = jnp.transpose(x, (0, 2, 1))
    xta = jnp.pad(xt, ((0, 0), (0, 0), (0, CP - CIN)))
    xtb = jnp.pad(xt, ((0, 0), (0, 0), (CIN, CP - 2 * CIN)))
    x8 = jnp.pad(x, ((0, 0), (0, 8 - CIN), (0, 0)))

    idx40 = _topk40(x8, xta)

    sel = jax.random.permutation(jax.random.key(42), KTOP)[:K]
    b1 = jnp.take(idx40, sel, axis=2)
    b2 = idx40[:, :, :K]
    allk = jnp.stack([b1, b2])
    allk = jnp.transpose(allk, (0, 1, 3, 2))
    allk = allk + (jnp.arange(B, dtype=jnp.int32) * N)[None, :, None, None]
    idx2d = allk.reshape(IDX_ROWS, GCHUNK)

    table = xta.reshape(B * N, CP)
    gath = _sc_gather(table, idx2d).reshape(2, B, RP, 128)

    eye8 = jnp.eye(8, dtype=jnp.float32)

    def big(w):
        return jnp.kron(eye8, w)

    w1s = jnp.stack([
        big(jnp.pad(p["conv1_w"].T, ((0, CP - 6), (0, 0)))),
        big(jnp.pad(p["conv1_2_w"].T, ((0, CP - 6), (0, 0)))),
    ])
    w2s = jnp.stack([big(p["conv2_w"].T), big(p["conv2_2_w"].T)])

    def catbig(w):
        return jnp.concatenate([big(w[:, :64].T), big(w[:, 64:].T)], axis=0)

    wcats = jnp.stack([catbig(p["conv_cat_w"]), catbig(p["conv_cat2_w"])])
    gnps = jnp.stack([
        jnp.stack([p["gn1_g"], p["gn1_b"], p["gn2_g"], p["gn2_b"],
                   p["gncat_g"], p["gncat_b"]]),
        jnp.stack([p["gn1_2_g"], p["gn1_2_b"], p["gn2_2_g"], p["gn2_2_b"],
                   p["gncat2_g"], p["gncat2_b"]]),
    ])
    se1s = jnp.stack([p["se1_w1"], p["se1_2_w1"]])
    se2s = jnp.stack([p["se1_w2"], p["se1_2_w2"]])

    xta_p = xta.reshape(B, NP, 128)
    xtb_p = xtb.reshape(B, NP, 128)
    bout = _branches(gath, xta_p, xtb_p, w1s, w2s, wcats, gnps, se1s, se2s)

    out_p = _final(bout, catbig(p["conv_add1_w"]),
                   p["gnadd1_g"].reshape(1, 64), p["gnadd1_b"].reshape(1, 64))
    return out_p.reshape(B, NP, 8, 64).transpose(0, 3, 1, 2).reshape(B, 64, N)


# speedup vs baseline: 1.0779x; 1.0779x over previous
import functools

import jax
import jax.numpy as jnp
from jax import lax
from jax.experimental import pallas as pl
from jax.experimental.pallas import tpu as pltpu
from jax.experimental.pallas import tpu_sc as plsc

B = 8
N = 2048
K = 20
CIN = 3
CP = 16
KTOP = 2 * K
ROWT = 512
EPS = 1e-5
NEG = float("-inf")

SC_NC = 2
SC_NS = 16
SC_NW = SC_NC * SC_NS
GCHUNK = 128
GB = 16
TOTAL_IDX = 2 * B * N * K
IDX_ROWS = TOTAL_IDX // GCHUNK
ROWS_PER_W = IDX_ROWS // SC_NW


def _topk_body(x8_ref, xt_ref, out_ref):
    xt = xt_ref[0]
    x0 = x8_ref[0, 0:1, :]
    x1 = x8_ref[0, 1:2, :]
    x2 = x8_ref[0, 2:3, :]

    xx_col = (x0 * x0 + x1 * x1) + x2 * x2
    t0 = xt[:, 0:1]
    t1 = xt[:, 1:2]
    t2 = xt[:, 2:3]
    xx_row = (t0 * t0 + t1 * t1) + t2 * t2
    def b16(v):
        return v.astype(jnp.bfloat16).astype(jnp.float32)
    g = (b16(t0) * b16(x0) + b16(t1) * b16(x1)) + b16(t2) * b16(x2)
    d = (-xx_col + 2.0 * g) - xx_row

    lanes = lax.broadcasted_iota(jnp.int32, (ROWT, N), 1)
    lanes40 = lax.broadcasted_iota(jnp.int32, (ROWT, KTOP), 1)
    acc0 = jnp.zeros((ROWT, KTOP), jnp.int32)

    def step(t, carry):
        dcur, acc = carry
        m = jnp.max(dcur, axis=1, keepdims=True)
        j = jnp.min(jnp.where(dcur == m, lanes, N), axis=1,
                    keepdims=True)
        acc = jnp.where(lanes40 == t, j, acc)
        dcur = jnp.where(lanes == j, NEG, dcur)
        return dcur, acc

    _, acc = lax.fori_loop(0, KTOP, step, (d, acc0))
    out_ref[0] = acc


def _topk40(x8, xt16):
    return pl.pallas_call(
        _topk_body,
        grid=(B, N // ROWT),
        in_specs=[
            pl.BlockSpec((1, 8, N), lambda b, i: (b, 0, 0)),
            pl.BlockSpec((1, ROWT, CP), lambda b, i: (b, i, 0)),
        ],
        out_specs=pl.BlockSpec((1, ROWT, KTOP), lambda b, i: (b, i, 0)),
        out_shape=jax.ShapeDtypeStruct((B, N, KTOP), jnp.int32),
    )(x8, xt16)


def _sc_gather(table, idx2d):
    mesh = plsc.VectorSubcoreMesh(core_axis_name="c", subcore_axis_name="s")

    @functools.partial(
        pl.kernel,
        mesh=mesh,
        compiler_params=pltpu.CompilerParams(use_tc_tiling_on_sc=False),
        out_type=jax.ShapeDtypeStruct((TOTAL_IDX, CP), jnp.float32),
        scratch_types=[
            pltpu.VMEM((ROWS_PER_W, GCHUNK), jnp.int32),
            pltpu.VMEM((GB * GCHUNK, CP), jnp.float32),
            pltpu.SemaphoreType.DMA,
        ],
    )
    def gk(table_hbm, idx_hbm, out_hbm, idx_v, rows_v, sem):
        wid = lax.axis_index("s") * SC_NC + lax.axis_index("c")
        base = wid * ROWS_PER_W
        pltpu.sync_copy(idx_hbm.at[pl.ds(base, ROWS_PER_W)], idx_v)

        def body(o, _):
            for b in range(GB):
                pltpu.async_copy(table_hbm.at[idx_v.at[o * GB + b]],
                                 rows_v.at[pl.ds(b * GCHUNK, GCHUNK)], sem)
            for b in range(GB):
                pltpu.make_async_copy(
                    table_hbm.at[idx_v.at[o * GB + b]],
                    rows_v.at[pl.ds(b * GCHUNK, GCHUNK)], sem).wait()
            pltpu.sync_copy(
                rows_v,
                out_hbm.at[pl.ds((base + o * GB) * GCHUNK, GB * GCHUNK)])
            return 0

        lax.fori_loop(0, ROWS_PER_W // GB, body, 0)

    return gk(table, idx2d)


def _pair_sum(v, axis):
    if axis == 1:
        left = jnp.concatenate([v[:, 1:], v[:, :1]], axis=1)
        right = jnp.concatenate([v[:, -1:], v[:, :-1]], axis=1)
    else:
        left = jnp.concatenate([v[1:, :], v[:1, :]], axis=0)
        right = jnp.concatenate([v[-1:, :], v[:-1, :]], axis=0)
    even = lax.broadcasted_iota(jnp.int32, v.shape, axis) % 2 == 0
    return v + jnp.where(even, left, right)


def _gn_lrelu(h, gamma, beta):
    n = h.shape[0] * 2
    s = jnp.sum(h, axis=0, keepdims=True)
    ss = jnp.sum(h * h, axis=0, keepdims=True)
    sg = _pair_sum(s, 1)
    ssg = _pair_sum(ss, 1)
    mean = sg / n
    var = ssg / n - mean * mean
    inv = lax.rsqrt(var + EPS)
    scale = gamma * inv
    shift = beta - mean * scale
    z = h * scale + shift
    return jnp.where(z >= 0, z, 0.2 * z)


def _fold512(v):
    v = v[:, :256] + v[:, 256:]
    v = v[:, :128] + v[:, 128:]
    return v[:, :64] + v[:, 64:]


def _tile8(v):
    return jnp.concatenate([v] * 8, axis=1)


def _gn_lrelu_packed(h, gamma, beta, nrows):
    n = nrows * 2
    s = _fold512(jnp.sum(h, axis=0, keepdims=True))
    ss = _fold512(jnp.sum(h * h, axis=0, keepdims=True))
    sg = _pair_sum(s, 1)
    ssg = _pair_sum(ss, 1)
    mean = sg / n
    var = ssg / n - mean * mean
    inv = lax.rsqrt(var + EPS)
    scale = gamma * inv
    shift = beta - mean * scale
    z = h * _tile8(scale) + _tile8(shift)
    return jnp.where(z >= 0, z, 0.2 * z)


NP = N // 8
RP = K * NP


def _branch_body(gath_ref, xta_ref, xtb_ref, w1_ref, w2_ref, wcat_ref,
                 gnp_ref, se1_ref, se2_ref, out_ref):
    g = gath_ref[0, 0]
    xa = xta_ref[0]
    xb = xtb_ref[0]

    e = (g.reshape(K, NP, 128) - xa[None] + xb[None]).reshape(RP, 128)
    h = jnp.dot(e, w1_ref[0], preferred_element_type=jnp.float32)
    h = _gn_lrelu_packed(h, gnp_ref[0, 0:1, :], gnp_ref[0, 1:2, :], K * N)
    h = jnp.dot(h, w2_ref[0], preferred_element_type=jnp.float32)
    h = _gn_lrelu_packed(h, gnp_ref[0, 2:3, :], gnp_ref[0, 3:4, :], K * N)

    h3 = h.reshape(K, NP, 512)
    mx = jnp.max(h3, axis=0)
    mn = jnp.sum(h3, axis=0) / K
    cat = jnp.concatenate([mx, mn], axis=1)
    c = jnp.dot(cat, wcat_ref[0], preferred_element_type=jnp.float32)
    c = _gn_lrelu_packed(c, gnp_ref[0, 4:5, :], gnp_ref[0, 5:6, :], N)

    y = _fold512(jnp.sum(c, axis=0, keepdims=True)) / N
    y1 = jax.nn.relu(lax.dot_general(y, se1_ref[0], (((1,), (1,)), ((), ())),
                                     preferred_element_type=jnp.float32))
    y2 = jax.nn.sigmoid(lax.dot_general(y1, se2_ref[0],
                                        (((1,), (1,)), ((), ())),
                                        preferred_element_type=jnp.float32))
    out_ref[0, 0] = c * _tile8(y2)


def _branches(gath, xta, xtb, w1s, w2s, wcats, gnps, se1s, se2s):
    return pl.pallas_call(
        _branch_body,
        grid=(2, B),
        in_specs=[
            pl.BlockSpec((1, 1, RP, 128), lambda r, b: (r, b, 0, 0)),
            pl.BlockSpec((1, NP, 128), lambda r, b: (b, 0, 0)),
            pl.BlockSpec((1, NP, 128), lambda r, b: (b, 0, 0)),
            pl.BlockSpec((1, 128, 512), lambda r, b: (r, 0, 0)),
            pl.BlockSpec((1, 512, 512), lambda r, b: (r, 0, 0)),
            pl.BlockSpec((1, 1024, 512), lambda r, b: (r, 0, 0)),
            pl.BlockSpec((1, 6, 64), lambda r, b: (r, 0, 0)),
            pl.BlockSpec((1, 4, 64), lambda r, b: (r, 0, 0)),
            pl.BlockSpec((1, 64, 4), lambda r, b: (r, 0, 0)),
        ],
        out_specs=pl.BlockSpec((1, 1, NP, 512), lambda r, b: (r, b, 0, 0)),
        out_shape=jax.ShapeDtypeStruct((2, B, NP, 512), jnp.float32),
    )(gath, xta, xtb, w1s, w2s, wcats, gnps, se1s, se2s)


def _final_body(b1_ref, b2_ref, wadd_ref, g_ref, bb_ref, out_ref):
    cat = jnp.concatenate([b1_ref[0, 0], b2_ref[0, 0]], axis=1)
    z = jnp.dot(cat, wadd_ref[...],
                preferred_element_type=jnp.float32)
    out_ref[0] = _gn_lrelu_packed(z, g_ref[...], bb_ref[...], N)


def _final(bout, waddbig, gg, gb):
    return pl.pallas_call(
        _final_body,
        grid=(B,),
        in_specs=[
            pl.BlockSpec((1, 1, NP, 512), lambda b: (0, b, 0, 0)),
            pl.BlockSpec((1, 1, NP, 512), lambda b: (1, b, 0, 0)),
            pl.BlockSpec((1024, 512), lambda b: (0, 0)),
            pl.BlockSpec((1, 64), lambda b: (0, 0)),
            pl.BlockSpec((1, 64), lambda b: (0, 0)),
        ],
        out_specs=pl.BlockSpec((1, NP, 512), lambda b: (b, 0, 0)),
        out_shape=jax.ShapeDtypeStruct((B, NP, 512), jnp.float32),
    )(bout, bout, waddbig, gg, gb)


def kernel(x, params):
    p = params
    xt = jnp.transpose(x, (0, 2, 1))
    xta = jnp.pad(xt, ((0, 0), (0, 0), (0, CP - CIN)))
    xtb = jnp.pad(xt, ((0, 0), (0, 0), (CIN, CP - 2 * CIN)))
    x8 = jnp.pad(x, ((0, 0), (0, 8 - CIN), (0, 0)))

    idx40 = _topk40(x8, xta)

    sel = jax.random.permutation(jax.random.key(42), KTOP)[:K]
    b1 = jnp.take(idx40, sel, axis=2)
    b2 = idx40[:, :, :K]
    allk = jnp.stack([b1, b2])
    allk = jnp.transpose(allk, (0, 1, 3, 2))
    allk = allk + (jnp.arange(B, dtype=jnp.int32) * N)[None, :, None, None]
    idx2d = allk.reshape(IDX_ROWS, GCHUNK)

    table = xta.reshape(B * N, CP)
    gath = _sc_gather(table, idx2d).reshape(2, B, RP, 128)

    eye8 = jnp.eye(8, dtype=jnp.float32)

    def big(w):
        return jnp.kron(eye8, w)

    w1s = jnp.stack([
        big(jnp.pad(p["conv1_w"].T, ((0, CP - 6), (0, 0)))),
        big(jnp.pad(p["conv1_2_w"].T, ((0, CP - 6), (0, 0)))),
    ])
    w2s = jnp.stack([big(p["conv2_w"].T), big(p["conv2_2_w"].T)])

    def catbig(w):
        return jnp.concatenate([big(w[:, :64].T), big(w[:, 64:].T)], axis=0)

    wcats = jnp.stack([catbig(p["conv_cat_w"]), catbig(p["conv_cat2_w"])])
    gnps = jnp.stack([
        jnp.stack([p["gn1_g"], p["gn1_b"], p["gn2_g"], p["gn2_b"],
                   p["gncat_g"], p["gncat_b"]]),
        jnp.stack([p["gn1_2_g"], p["gn1_2_b"], p["gn2_2_g"], p["gn2_2_b"],
                   p["gncat2_g"], p["gncat2_b"]]),
    ])
    se1s = jnp.stack([p["se1_w1"], p["se1_2_w1"]])
    se2s = jnp.stack([p["se1_w2"], p["se1_2_w2"]])

    xta_p = xta.reshape(B, NP, 128)
    xtb_p = xtb.reshape(B, NP, 128)
    bout = _branches(gath, xta_p, xtb_p, w1s, w2s, wcats, gnps, se1s, se2s)

    out_p = _final(bout, catbig(p["conv_add1_w"]),
                   p["gnadd1_g"].reshape(1, 64), p["gnadd1_b"].reshape(1, 64))
    return out_p.reshape(B, NP, 8, 64).transpose(0, 3, 1, 2).reshape(B, 64, N)
